# baseline (speedup 1.0000x reference)
import functools

import jax
import jax.numpy as jnp
from jax import lax
from jax.experimental import pallas as pl
from jax.experimental.pallas import tpu as pltpu
from jax.experimental.pallas import tpu_sc as plsc

B = 16384
D_MLP = 128
D_GMF = 32
NROWS = 100000

_info = plsc.get_sparse_core_info()
NC, NS = _info.num_cores, _info.num_subcores
NW = NC * NS
BPW = B // NW

_sc_mesh = plsc.VectorSubcoreMesh(core_axis_name="c", subcore_axis_name="s")


@functools.partial(
    pl.kernel,
    mesh=_sc_mesh,
    out_type=[
        jax.ShapeDtypeStruct((B, D_MLP), jnp.float32),
        jax.ShapeDtypeStruct((B, D_MLP), jnp.float32),
    ],
    scratch_types=[
        pltpu.VMEM((BPW,), jnp.int32),
        pltpu.VMEM((BPW,), jnp.int32),
        pltpu.VMEM((BPW, D_MLP), jnp.float32),
        pltpu.SemaphoreType.DMA,
    ],
)
def _sc_gather_mlp(user_hbm, item_hbm, mue_hbm, mie_hbm,
                   mu_out, mi_out, idx_u, idx_i, buf, sem):
    wid = lax.axis_index("s") * NC + lax.axis_index("c")
    base = wid * BPW
    pltpu.sync_copy(user_hbm.at[pl.ds(base, BPW)], idx_u)
    pltpu.sync_copy(item_hbm.at[pl.ds(base, BPW)], idx_i)
    pltpu.async_copy(mue_hbm.at[idx_u], buf, sem).wait()
    pltpu.sync_copy(buf, mu_out.at[pl.ds(base, BPW)])
    pltpu.async_copy(mie_hbm.at[idx_i], buf, sem).wait()
    pltpu.sync_copy(buf, mi_out.at[pl.ds(base, BPW)])


@functools.partial(
    pl.kernel,
    mesh=_sc_mesh,
    out_type=[
        jax.ShapeDtypeStruct((B, 128), jnp.float32),
        jax.ShapeDtypeStruct((B, 128), jnp.float32),
    ],
    scratch_types=[
        pltpu.VMEM((BPW,), jnp.int32),
        pltpu.VMEM((BPW,), jnp.int32),
        pltpu.VMEM((BPW, 128), jnp.float32),
        pltpu.SemaphoreType.DMA,
    ],
)
def _sc_gather_gmf(user_hbm, item_hbm, comb_hbm,
                   gu_out, gi_out, idx_u, idx_i, buf, sem):
    wid = lax.axis_index("s") * NC + lax.axis_index("c")
    base = wid * BPW
    pltpu.sync_copy(user_hbm.at[pl.ds(base, BPW)], idx_u)
    pltpu.sync_copy(item_hbm.at[pl.ds(base, BPW)], idx_i)
    pltpu.async_copy(comb_hbm.at[idx_u], buf, sem).wait()
    pltpu.sync_copy(buf, gu_out.at[pl.ds(base, BPW)])
    pltpu.async_copy(comb_hbm.at[idx_i], buf, sem).wait()
    pltpu.sync_copy(buf, gi_out.at[pl.ds(base, BPW)])


RPK = 5000


def _pack_body(a_ref, b_ref, o_ref):
    a = a_ref[...]
    b = b_ref[...]
    o_ref[:, 0:D_GMF] = a
    o_ref[:, D_GMF:2 * D_GMF] = b
    o_ref[:, 2 * D_GMF:3 * D_GMF] = a
    o_ref[:, 3 * D_GMF:] = b


def _pack(gue, gie):
    grid = NROWS // RPK
    return pl.pallas_call(
        _pack_body,
        grid=(grid,),
        in_specs=[
            pl.BlockSpec((RPK, D_GMF), lambda i: (i, 0)),
            pl.BlockSpec((RPK, D_GMF), lambda i: (i, 0)),
        ],
        out_specs=pl.BlockSpec((RPK, 128), lambda i: (i, 0)),
        out_shape=jax.ShapeDtypeStruct((NROWS, 128), jnp.float32),
    )(gue, gie)


BLK = 2048


def _mlp_body(mu, mi, gub, gib, w1a, w1b, b1, w2, b2, w3, b3,
              wpg, wpx, bp, out):
    x = jnp.dot(mu[...], w1a[...], preferred_element_type=jnp.float32)
    x = x + jnp.dot(mi[...], w1b[...], preferred_element_type=jnp.float32)
    x = jnp.maximum(x + b1[...], 0.0)
    x = jnp.maximum(
        jnp.dot(x, w2[...], preferred_element_type=jnp.float32) + b2[...], 0.0)
    x = jnp.maximum(
        jnp.dot(x, w3[...], preferred_element_type=jnp.float32) + b3[...], 0.0)
    g = gub[:, 0:D_GMF] * gib[:, D_GMF:2 * D_GMF]
    logit = (jnp.sum(g * wpg[...], axis=1)
             + jnp.sum(x * wpx[...], axis=1) + bp[0, 0])
    out[...] = 1.0 / (1.0 + jnp.exp(-logit))


def _run_mlp(mu, mi, gub, gib,
             w1a, w1b, b1, w2, b2, w3, b3, wpg, wpx, bp):
    grid = B // BLK
    row = lambda i: (i, 0)
    full = lambda i: (0, 0)
    return pl.pallas_call(
        _mlp_body,
        grid=(grid,),
        in_specs=[
            pl.BlockSpec((BLK, D_MLP), row),
            pl.BlockSpec((BLK, D_MLP), row),
            pl.BlockSpec((BLK, 128), row),
            pl.BlockSpec((BLK, 128), row),
            pl.BlockSpec((D_MLP, 128), full),
            pl.BlockSpec((D_MLP, 128), full),
            pl.BlockSpec((1, 128), full),
            pl.BlockSpec((128, 64), full),
            pl.BlockSpec((1, 64), full),
            pl.BlockSpec((64, 32), full),
            pl.BlockSpec((1, 32), full),
            pl.BlockSpec((1, 32), full),
            pl.BlockSpec((1, 32), full),
            pl.BlockSpec((1, 1), full),
        ],
        out_specs=pl.BlockSpec((BLK,), lambda i: (i,)),
        out_shape=jax.ShapeDtypeStruct((B,), jnp.float32),
    )(mu, mi, gub, gib, w1a, w1b, b1, w2, b2, w3, b3, wpg, wpx, bp)


def kernel(user, item, gmf_user_emb, gmf_item_emb, mlp_user_emb, mlp_item_emb,
           W1, b1, W2, b2, W3, b3, Wp, bp):
    user = user.astype(jnp.int32)
    item = item.astype(jnp.int32)
    comb = _pack(gmf_user_emb, gmf_item_emb)
    mu, mi = _sc_gather_mlp(user, item, mlp_user_emb, mlp_item_emb)
    gub, gib = _sc_gather_gmf(user, item, comb)
    w1t = W1.T
    w1a, w1b = w1t[:D_MLP], w1t[D_MLP:]
    wpg = Wp[:, :D_GMF]
    wpx = Wp[:, D_GMF:]
    return _run_mlp(mu, mi, gub, gib,
                    w1a, w1b, b1.reshape(1, -1),
                    W2.T, b2.reshape(1, -1), W3.T, b3.reshape(1, -1),
                    wpg, wpx, bp.reshape(1, 1))


# speedup vs baseline: 1.2811x; 1.2811x over previous
import functools

import jax
import jax.numpy as jnp
from jax import lax
from jax.experimental import pallas as pl
from jax.experimental.pallas import tpu as pltpu
from jax.experimental.pallas import tpu_sc as plsc

B = 16384
D_MLP = 128
D_GMF = 32
NROWS = 100000

_info = plsc.get_sparse_core_info()
NC, NS = _info.num_cores, _info.num_subcores
NW = NC * NS
BPW = B // NW

_sc_mesh = plsc.VectorSubcoreMesh(core_axis_name="c", subcore_axis_name="s")


@functools.partial(
    pl.kernel,
    mesh=_sc_mesh,
    out_type=[
        jax.ShapeDtypeStruct((B, D_MLP), jnp.float32),
        jax.ShapeDtypeStruct((B, D_MLP), jnp.float32),
        jax.ShapeDtypeStruct((B, D_GMF), jnp.float32),
        jax.ShapeDtypeStruct((B, D_GMF), jnp.float32),
    ],
    scratch_types=[
        pltpu.VMEM((BPW,), jnp.int32),
        pltpu.VMEM((BPW,), jnp.int32),
        pltpu.VMEM((BPW // 2, D_MLP), jnp.float32),
        pltpu.VMEM((BPW, D_GMF), jnp.float32),
        pltpu.SemaphoreType.DMA,
        pltpu.SemaphoreType.DMA,
    ],
)
def _sc_gather(user_hbm, item_hbm, mue_hbm, mie_hbm, gue_hbm, gie_hbm,
               mu_out, mi_out, gu_out, gi_out, idx_u, idx_i, buf, bg,
               sem, gsem):
    wid = lax.axis_index("s") * NC + lax.axis_index("c")
    base = wid * BPW
    pltpu.sync_copy(user_hbm.at[pl.ds(base, BPW)], idx_u)
    pltpu.sync_copy(item_hbm.at[pl.ds(base, BPW)], idx_i)

    RCH = 16
    nch = BPW // RCH

    def fire_chunk(c, idx, tbl):
        v = idx[pl.ds(c * RCH, RCH)]
        for r in range(RCH):
            pltpu.async_copy(tbl.at[pl.ds(v[r], 1)],
                             bg.at[pl.ds(c * RCH + r, 1)], gsem)

    def drain_chunk(tbl):
        pltpu.make_async_copy(tbl.at[pl.ds(0, RCH)],
                              bg.at[pl.ds(0, RCH)], gsem).wait()

    def row_gather(idx, tbl, out):
        fire_chunk(0, idx, tbl)

        def body(c, _):
            fire_chunk(c, idx, tbl)
            drain_chunk(tbl)
            return ()
        lax.fori_loop(1, nch, body, ())
        drain_chunk(tbl)
        pltpu.sync_copy(bg, out.at[pl.ds(base, BPW)])

    row_gather(idx_u, gue_hbm, gu_out)
    row_gather(idx_i, gie_hbm, gi_out)

    HB = BPW // 2
    for h in range(2):
        pltpu.async_copy(mue_hbm.at[idx_u.at[pl.ds(h * HB, HB)]],
                         buf, sem).wait()
        pltpu.sync_copy(buf, mu_out.at[pl.ds(base + h * HB, HB)])
    for h in range(2):
        pltpu.async_copy(mie_hbm.at[idx_i.at[pl.ds(h * HB, HB)]],
                         buf, sem).wait()
        pltpu.sync_copy(buf, mi_out.at[pl.ds(base + h * HB, HB)])


BLK = 2048


def _mlp_body(mu, mi, gub, gib, w1a, w1b, b1, w2, b2, w3, b3,
              wpg, wpx, bp, out):
    x = jnp.dot(mu[...], w1a[...], preferred_element_type=jnp.float32)
    x = x + jnp.dot(mi[...], w1b[...], preferred_element_type=jnp.float32)
    x = jnp.maximum(x + b1[...], 0.0)
    x = jnp.maximum(
        jnp.dot(x, w2[...], preferred_element_type=jnp.float32) + b2[...], 0.0)
    x = jnp.maximum(
        jnp.dot(x, w3[...], preferred_element_type=jnp.float32) + b3[...], 0.0)
    g = gub[...] * gib[...]
    logit = (jnp.sum(g * wpg[...], axis=1)
             + jnp.sum(x * wpx[...], axis=1) + bp[0, 0])
    out[...] = 1.0 / (1.0 + jnp.exp(-logit))


def _run_mlp(mu, mi, gub, gib,
             w1a, w1b, b1, w2, b2, w3, b3, wpg, wpx, bp):
    grid = B // BLK
    row = lambda i: (i, 0)
    full = lambda i: (0, 0)
    return pl.pallas_call(
        _mlp_body,
        grid=(grid,),
        in_specs=[
            pl.BlockSpec((BLK, D_MLP), row),
            pl.BlockSpec((BLK, D_MLP), row),
            pl.BlockSpec((BLK, D_GMF), row),
            pl.BlockSpec((BLK, D_GMF), row),
            pl.BlockSpec((D_MLP, 128), full),
            pl.BlockSpec((D_MLP, 128), full),
            pl.BlockSpec((1, 128), full),
            pl.BlockSpec((128, 64), full),
            pl.BlockSpec((1, 64), full),
            pl.BlockSpec((64, 32), full),
            pl.BlockSpec((1, 32), full),
            pl.BlockSpec((1, 32), full),
            pl.BlockSpec((1, 32), full),
            pl.BlockSpec((1, 1), full),
        ],
        out_specs=pl.BlockSpec((BLK,), lambda i: (i,)),
        out_shape=jax.ShapeDtypeStruct((B,), jnp.float32),
    )(mu, mi, gub, gib, w1a, w1b, b1, w2, b2, w3, b3, wpg, wpx, bp)


def kernel(user, item, gmf_user_emb, gmf_item_emb, mlp_user_emb, mlp_item_emb,
           W1, b1, W2, b2, W3, b3, Wp, bp):
    user = user.astype(jnp.int32)
    item = item.astype(jnp.int32)
    mu, mi, gub, gib = _sc_gather(user, item, mlp_user_emb, mlp_item_emb,
                                  gmf_user_emb, gmf_item_emb)
    w1t = W1.T
    w1a, w1b = w1t[:D_MLP], w1t[D_MLP:]
    wpg = Wp[:, :D_GMF]
    wpx = Wp[:, D_GMF:]
    return _run_mlp(mu, mi, gub, gib,
                    w1a, w1b, b1.reshape(1, -1),
                    W2.T, b2.reshape(1, -1), W3.T, b3.reshape(1, -1),
                    wpg, wpx, bp.reshape(1, 1))


# speedup vs baseline: 1.3016x; 1.0160x over previous
import functools

import jax
import jax.numpy as jnp
from jax import lax
from jax.experimental import pallas as pl
from jax.experimental.pallas import tpu as pltpu
from jax.experimental.pallas import tpu_sc as plsc

B = 16384
D_MLP = 128
D_GMF = 32
NROWS = 100000

_info = plsc.get_sparse_core_info()
NC, NS = _info.num_cores, _info.num_subcores
NW = NC * NS
BPW = B // NW

_sc_mesh = plsc.VectorSubcoreMesh(core_axis_name="c", subcore_axis_name="s")


@functools.partial(
    pl.kernel,
    mesh=_sc_mesh,
    out_type=[
        jax.ShapeDtypeStruct((B, D_MLP), jnp.float32),
        jax.ShapeDtypeStruct((B, D_MLP), jnp.float32),
        jax.ShapeDtypeStruct((B, D_GMF), jnp.float32),
        jax.ShapeDtypeStruct((B, D_GMF), jnp.float32),
    ],
    scratch_types=[
        pltpu.VMEM((BPW,), jnp.int32),
        pltpu.VMEM((BPW,), jnp.int32),
        pltpu.VMEM((BPW // 2, D_MLP), jnp.float32),
        pltpu.VMEM((BPW, D_GMF), jnp.float32),
        pltpu.SemaphoreType.DMA,
        pltpu.SemaphoreType.DMA,
    ],
)
def _sc_gather(user_hbm, item_hbm, mue_hbm, mie_hbm, gue_hbm, gie_hbm,
               mu_out, mi_out, gu_out, gi_out, idx_u, idx_i, buf, bg,
               sem, gsem):
    wid = lax.axis_index("s") * NC + lax.axis_index("c")
    base = wid * BPW
    pltpu.sync_copy(user_hbm.at[pl.ds(base, BPW)], idx_u)
    pltpu.sync_copy(item_hbm.at[pl.ds(base, BPW)], idx_i)

    RCH = 16
    nch = BPW // RCH

    def fire_chunk(c, idx, tbl):
        v = idx[pl.ds(c * RCH, RCH)]
        for r in range(RCH):
            pltpu.async_copy(tbl.at[pl.ds(v[r], 1)],
                             bg.at[pl.ds(c * RCH + r, 1)], gsem)

    def drain_chunk(tbl):
        pltpu.make_async_copy(tbl.at[pl.ds(0, RCH)],
                              bg.at[pl.ds(0, RCH)], gsem).wait()

    def row_fires(idx, tbl):
        fire_chunk(0, idx, tbl)

        def body(c, _):
            fire_chunk(c, idx, tbl)
            drain_chunk(tbl)
            return ()
        lax.fori_loop(1, nch, body, ())
        drain_chunk(tbl)

    HB = BPW // 2
    cp1 = pltpu.async_copy(mue_hbm.at[idx_u.at[pl.ds(0, HB)]], buf, sem)
    row_fires(idx_u, gue_hbm)
    pltpu.sync_copy(bg, gu_out.at[pl.ds(base, BPW)])
    cp1.wait()
    pltpu.sync_copy(buf, mu_out.at[pl.ds(base, HB)])

    cp2 = pltpu.async_copy(mue_hbm.at[idx_u.at[pl.ds(HB, HB)]], buf, sem)
    row_fires(idx_i, gie_hbm)
    pltpu.sync_copy(bg, gi_out.at[pl.ds(base, BPW)])
    cp2.wait()
    pltpu.sync_copy(buf, mu_out.at[pl.ds(base + HB, HB)])

    for h in range(2):
        pltpu.async_copy(mie_hbm.at[idx_i.at[pl.ds(h * HB, HB)]],
                         buf, sem).wait()
        pltpu.sync_copy(buf, mi_out.at[pl.ds(base + h * HB, HB)])


BLK = 2048


def _mlp_body(mu, mi, gub, gib, w1a, w1b, b1, w2, b2, w3, b3,
              wpg, wpx, bp, out):
    x = jnp.dot(mu[...], w1a[...], preferred_element_type=jnp.float32)
    x = x + jnp.dot(mi[...], w1b[...], preferred_element_type=jnp.float32)
    x = jnp.maximum(x + b1[...], 0.0)
    x = jnp.maximum(
        jnp.dot(x, w2[...], preferred_element_type=jnp.float32) + b2[...], 0.0)
    x = jnp.maximum(
        jnp.dot(x, w3[...], preferred_element_type=jnp.float32) + b3[...], 0.0)
    g = gub[...] * gib[...]
    logit = (jnp.sum(g * wpg[...], axis=1)
             + jnp.sum(x * wpx[...], axis=1) + bp[0, 0])
    out[...] = 1.0 / (1.0 + jnp.exp(-logit))


def _run_mlp(mu, mi, gub, gib,
             w1a, w1b, b1, w2, b2, w3, b3, wpg, wpx, bp):
    grid = B // BLK
    row = lambda i: (i, 0)
    full = lambda i: (0, 0)
    return pl.pallas_call(
        _mlp_body,
        grid=(grid,),
        in_specs=[
            pl.BlockSpec((BLK, D_MLP), row),
            pl.BlockSpec((BLK, D_MLP), row),
            pl.BlockSpec((BLK, D_GMF), row),
            pl.BlockSpec((BLK, D_GMF), row),
            pl.BlockSpec((D_MLP, 128), full),
            pl.BlockSpec((D_MLP, 128), full),
            pl.BlockSpec((1, 128), full),
            pl.BlockSpec((128, 64), full),
            pl.BlockSpec((1, 64), full),
            pl.BlockSpec((64, 32), full),
            pl.BlockSpec((1, 32), full),
            pl.BlockSpec((1, 32), full),
            pl.BlockSpec((1, 32), full),
            pl.BlockSpec((1, 1), full),
        ],
        out_specs=pl.BlockSpec((BLK,), lambda i: (i,)),
        out_shape=jax.ShapeDtypeStruct((B,), jnp.float32),
    )(mu, mi, gub, gib, w1a, w1b, b1, w2, b2, w3, b3, wpg, wpx, bp)


def kernel(user, item, gmf_user_emb, gmf_item_emb, mlp_user_emb, mlp_item_emb,
           W1, b1, W2, b2, W3, b3, Wp, bp):
    user = user.astype(jnp.int32)
    item = item.astype(jnp.int32)
    mu, mi, gub, gib = _sc_gather(user, item, mlp_user_emb, mlp_item_emb,
                                  gmf_user_emb, gmf_item_emb)
    w1t = W1.T
    w1a, w1b = w1t[:D_MLP], w1t[D_MLP:]
    wpg = Wp[:, :D_GMF]
    wpx = Wp[:, D_GMF:]
    return _run_mlp(mu, mi, gub, gib,
                    w1a, w1b, b1.reshape(1, -1),
                    W2.T, b2.reshape(1, -1), W3.T, b3.reshape(1, -1),
                    wpg, wpx, bp.reshape(1, 1))


# speedup vs baseline: 1.4467x; 1.1115x over previous
import functools

import jax
import jax.numpy as jnp
from jax import lax
from jax.experimental import pallas as pl
from jax.experimental.pallas import tpu as pltpu
from jax.experimental.pallas import tpu_sc as plsc

B = 16384
D_MLP = 128
D_GMF = 32
NROWS = 100000

_info = plsc.get_sparse_core_info()
NC, NS = _info.num_cores, _info.num_subcores
NW = NC * NS
BPW = B // NW

_sc_mesh = plsc.VectorSubcoreMesh(core_axis_name="c", subcore_axis_name="s")


@functools.partial(
    pl.kernel,
    mesh=_sc_mesh,
    out_type=[
        jax.ShapeDtypeStruct((B, D_MLP), jnp.float32),
        jax.ShapeDtypeStruct((B, D_MLP), jnp.float32),
        jax.ShapeDtypeStruct((B, D_GMF), jnp.float32),
        jax.ShapeDtypeStruct((B, D_GMF), jnp.float32),
    ],
    scratch_types=[
        pltpu.VMEM((BPW,), jnp.int32),
        pltpu.VMEM((BPW,), jnp.int32),
        pltpu.VMEM((BPW // 2, D_MLP), jnp.float32),
        pltpu.VMEM((BPW, D_GMF), jnp.float32),
        pltpu.SemaphoreType.DMA,
        pltpu.SemaphoreType.DMA,
    ],
)
def _sc_gather(user_hbm, item_hbm, mue_hbm, mie_hbm, gue_hbm, gie_hbm,
               mu_out, mi_out, gu_out, gi_out, idx_u, idx_i, buf, bg,
               sem, gsem):
    wid = lax.axis_index("s") * NC + lax.axis_index("c")
    base = wid * BPW
    pltpu.sync_copy(user_hbm.at[pl.ds(base, BPW)], idx_u)
    pltpu.sync_copy(item_hbm.at[pl.ds(base, BPW)], idx_i)

    RCH = 16
    nch = BPW // RCH

    def fire_chunk(c, idx, tbl):
        v = idx[pl.ds(c * RCH, RCH)]
        for r in range(RCH):
            pltpu.async_copy(tbl.at[pl.ds(v[r], 1)],
                             bg.at[pl.ds(c * RCH + r, 1)], gsem)

    def row_fires(idx, tbl):
        def body(c, _):
            fire_chunk(c, idx, tbl)
            return ()
        lax.fori_loop(0, nch, body, ())
        pltpu.make_async_copy(tbl.at[pl.ds(0, BPW)], bg, gsem).wait()

    HB = BPW // 2
    cp1 = pltpu.async_copy(mue_hbm.at[idx_u.at[pl.ds(0, HB)]], buf, sem)
    row_fires(idx_u, gue_hbm)
    pltpu.sync_copy(bg, gu_out.at[pl.ds(base, BPW)])
    cp1.wait()
    pltpu.sync_copy(buf, mu_out.at[pl.ds(base, HB)])

    cp2 = pltpu.async_copy(mue_hbm.at[idx_u.at[pl.ds(HB, HB)]], buf, sem)
    row_fires(idx_i, gie_hbm)
    pltpu.sync_copy(bg, gi_out.at[pl.ds(base, BPW)])
    cp2.wait()
    pltpu.sync_copy(buf, mu_out.at[pl.ds(base + HB, HB)])

    for h in range(2):
        pltpu.async_copy(mie_hbm.at[idx_i.at[pl.ds(h * HB, HB)]],
                         buf, sem).wait()
        pltpu.sync_copy(buf, mi_out.at[pl.ds(base + h * HB, HB)])


BLK = 2048


def _mlp_body(mu, mi, gub, gib, w1a, w1b, b1, w2, b2, w3, b3,
              wpg, wpx, bp, out):
    x = jnp.dot(mu[...], w1a[...], preferred_element_type=jnp.float32)
    x = x + jnp.dot(mi[...], w1b[...], preferred_element_type=jnp.float32)
    x = jnp.maximum(x + b1[...], 0.0)
    x = jnp.maximum(
        jnp.dot(x, w2[...], preferred_element_type=jnp.float32) + b2[...], 0.0)
    x = jnp.maximum(
        jnp.dot(x, w3[...], preferred_element_type=jnp.float32) + b3[...], 0.0)
    g = gub[...] * gib[...]
    logit = (jnp.sum(g * wpg[...], axis=1)
             + jnp.sum(x * wpx[...], axis=1) + bp[0, 0])
    out[...] = 1.0 / (1.0 + jnp.exp(-logit))


def _run_mlp(mu, mi, gub, gib,
             w1a, w1b, b1, w2, b2, w3, b3, wpg, wpx, bp):
    grid = B // BLK
    row = lambda i: (i, 0)
    full = lambda i: (0, 0)
    return pl.pallas_call(
        _mlp_body,
        grid=(grid,),
        in_specs=[
            pl.BlockSpec((BLK, D_MLP), row),
            pl.BlockSpec((BLK, D_MLP), row),
            pl.BlockSpec((BLK, D_GMF), row),
            pl.BlockSpec((BLK, D_GMF), row),
            pl.BlockSpec((D_MLP, 128), full),
            pl.BlockSpec((D_MLP, 128), full),
            pl.BlockSpec((1, 128), full),
            pl.BlockSpec((128, 64), full),
            pl.BlockSpec((1, 64), full),
            pl.BlockSpec((64, 32), full),
            pl.BlockSpec((1, 32), full),
            pl.BlockSpec((1, 32), full),
            pl.BlockSpec((1, 32), full),
            pl.BlockSpec((1, 1), full),
        ],
        out_specs=pl.BlockSpec((BLK,), lambda i: (i,)),
        out_shape=jax.ShapeDtypeStruct((B,), jnp.float32),
    )(mu, mi, gub, gib, w1a, w1b, b1, w2, b2, w3, b3, wpg, wpx, bp)


def kernel(user, item, gmf_user_emb, gmf_item_emb, mlp_user_emb, mlp_item_emb,
           W1, b1, W2, b2, W3, b3, Wp, bp):
    user = user.astype(jnp.int32)
    item = item.astype(jnp.int32)
    mu, mi, gub, gib = _sc_gather(user, item, mlp_user_emb, mlp_item_emb,
                                  gmf_user_emb, gmf_item_emb)
    w1t = W1.T
    w1a, w1b = w1t[:D_MLP], w1t[D_MLP:]
    wpg = Wp[:, :D_GMF]
    wpx = Wp[:, D_GMF:]
    return _run_mlp(mu, mi, gub, gib,
                    w1a, w1b, b1.reshape(1, -1),
                    W2.T, b2.reshape(1, -1), W3.T, b3.reshape(1, -1),
                    wpg, wpx, bp.reshape(1, 1))


# speedup vs baseline: 1.4653x; 1.0129x over previous
import functools

import jax
import jax.numpy as jnp
from jax import lax
from jax.experimental import pallas as pl
from jax.experimental.pallas import tpu as pltpu
from jax.experimental.pallas import tpu_sc as plsc

B = 16384
D_MLP = 128
D_GMF = 32
NROWS = 100000

_info = plsc.get_sparse_core_info()
NC, NS = _info.num_cores, _info.num_subcores
NW = NC * NS
BPW = B // NW
QR = BPW // 4
RCH = 16

_sc_mesh = plsc.VectorSubcoreMesh(core_axis_name="c", subcore_axis_name="s")


@functools.partial(
    pl.kernel,
    mesh=_sc_mesh,
    out_type=[
        jax.ShapeDtypeStruct((B, D_MLP), jnp.float32),
        jax.ShapeDtypeStruct((B, D_MLP), jnp.float32),
        jax.ShapeDtypeStruct((B, D_GMF), jnp.float32),
        jax.ShapeDtypeStruct((B, D_GMF), jnp.float32),
    ],
    scratch_types=[
        pltpu.VMEM((BPW,), jnp.int32),
        pltpu.VMEM((BPW,), jnp.int32),
        pltpu.VMEM((QR, D_MLP), jnp.float32),
        pltpu.VMEM((QR, D_MLP), jnp.float32),
        pltpu.VMEM((BPW, D_GMF), jnp.float32),
        pltpu.SemaphoreType.DMA,
        pltpu.SemaphoreType.DMA,
        pltpu.SemaphoreType.DMA,
        pltpu.SemaphoreType.DMA,
        pltpu.SemaphoreType.DMA,
    ],
)
def _sc_gather(user_hbm, item_hbm, mue_hbm, mie_hbm, gue_hbm, gie_hbm,
               mu_out, mi_out, gu_out, gi_out, idx_u, idx_i, buf_a, buf_b,
               bg, gsem_a, gsem_b, osem_a, osem_b, rsem):
    wid = lax.axis_index("s") * NC + lax.axis_index("c")
    base = wid * BPW
    pltpu.sync_copy(user_hbm.at[pl.ds(base, BPW)], idx_u)
    pltpu.sync_copy(item_hbm.at[pl.ds(base, BPW)], idx_i)

    bufs = (buf_a, buf_b)
    gsems = (gsem_a, gsem_b)
    osems = (osem_a, osem_b)
    nch = BPW // RCH

    def fire_rows(idx, tbl):
        def body(c, _):
            v = idx[pl.ds(c * RCH, RCH)]
            for r in range(RCH):
                pltpu.async_copy(tbl.at[pl.ds(v[r], 1)],
                                 bg.at[pl.ds(c * RCH + r, 1)], rsem)
            return ()
        lax.fori_loop(0, nch, body, ())

    tables = ((mue_hbm, mu_out, idx_u), (mie_hbm, mi_out, idx_i))

    for q in range(8):
        t, k = divmod(q, 4)
        tbl, out, idx = tables[t]
        b = q % 2
        if q >= 2:
            pltpu.make_async_copy(bufs[b], out.at[pl.ds(base, QR)],
                                  osems[b]).wait()
        pltpu.async_copy(tbl.at[idx.at[pl.ds(k * QR, QR)]], bufs[b],
                         gsems[b])
        if q == 0:
            fire_rows(idx_u, gue_hbm)
        elif q == 4:
            pltpu.make_async_copy(gue_hbm.at[pl.ds(0, BPW)], bg, rsem).wait()
            pltpu.sync_copy(bg, gu_out.at[pl.ds(base, BPW)])
            fire_rows(idx_i, gie_hbm)
        if q >= 1:
            pt, pk = divmod(q - 1, 4)
            pb = (q - 1) % 2
            ptbl, pout, pidx = tables[pt]
            pltpu.make_async_copy(ptbl.at[pidx.at[pl.ds(pk * QR, QR)]],
                                  bufs[pb], gsems[pb]).wait()
            pltpu.async_copy(bufs[pb],
                             pout.at[pl.ds(base + pk * QR, QR)], osems[pb])

    pltpu.make_async_copy(mie_hbm.at[idx_i.at[pl.ds(3 * QR, QR)]],
                          bufs[1], gsems[1]).wait()
    pltpu.sync_copy(bufs[1], mi_out.at[pl.ds(base + 3 * QR, QR)])
    pltpu.make_async_copy(bufs[0], mi_out.at[pl.ds(base, QR)],
                          osems[0]).wait()

    pltpu.make_async_copy(gie_hbm.at[pl.ds(0, BPW)], bg, rsem).wait()
    pltpu.sync_copy(bg, gi_out.at[pl.ds(base, BPW)])


BLK = 2048


def _mlp_body(mu, mi, gu, gi, w1a, w1b, b1, w2, b2, w3, b3,
              wpg, wpx, bp, out):
    x = jnp.dot(mu[...], w1a[...], preferred_element_type=jnp.float32)
    x = x + jnp.dot(mi[...], w1b[...], preferred_element_type=jnp.float32)
    x = jnp.maximum(x + b1[...], 0.0)
    x = jnp.maximum(
        jnp.dot(x, w2[...], preferred_element_type=jnp.float32) + b2[...], 0.0)
    x = jnp.maximum(
        jnp.dot(x, w3[...], preferred_element_type=jnp.float32) + b3[...], 0.0)
    g = gu[...] * gi[...]
    logit = (jnp.sum(g * wpg[...], axis=1)
             + jnp.sum(x * wpx[...], axis=1) + bp[0, 0])
    out[...] = 1.0 / (1.0 + jnp.exp(-logit))


def _run_mlp(mu, mi, gu, gi, w1a, w1b, b1, w2, b2, w3, b3, wpg, wpx, bp):
    grid = B // BLK
    row = lambda i: (i, 0)
    full = lambda i: (0, 0)
    return pl.pallas_call(
        _mlp_body,
        grid=(grid,),
        in_specs=[
            pl.BlockSpec((BLK, D_MLP), row),
            pl.BlockSpec((BLK, D_MLP), row),
            pl.BlockSpec((BLK, D_GMF), row),
            pl.BlockSpec((BLK, D_GMF), row),
            pl.BlockSpec((D_MLP, 128), full),
            pl.BlockSpec((D_MLP, 128), full),
            pl.BlockSpec((1, 128), full),
            pl.BlockSpec((128, 64), full),
            pl.BlockSpec((1, 64), full),
            pl.BlockSpec((64, 32), full),
            pl.BlockSpec((1, 32), full),
            pl.BlockSpec((1, 32), full),
            pl.BlockSpec((1, 32), full),
            pl.BlockSpec((1, 1), full),
        ],
        out_specs=pl.BlockSpec((BLK,), lambda i: (i,)),
        out_shape=jax.ShapeDtypeStruct((B,), jnp.float32),
    )(mu, mi, gu, gi, w1a, w1b, b1, w2, b2, w3, b3, wpg, wpx, bp)


def kernel(user, item, gmf_user_emb, gmf_item_emb, mlp_user_emb, mlp_item_emb,
           W1, b1, W2, b2, W3, b3, Wp, bp):
    user = user.astype(jnp.int32)
    item = item.astype(jnp.int32)
    mu, mi, gu, gi = _sc_gather(user, item, mlp_user_emb, mlp_item_emb,
                                gmf_user_emb, gmf_item_emb)
    w1t = W1.T
    w1a, w1b = w1t[:D_MLP], w1t[D_MLP:]
    wpg = Wp[:, :D_GMF]
    wpx = Wp[:, D_GMF:]
    return _run_mlp(mu, mi, gu, gi, w1a, w1b, b1.reshape(1, -1),
                    W2.T, b2.reshape(1, -1), W3.T, b3.reshape(1, -1),
                    wpg, wpx, bp.reshape(1, 1))
